# baseline (speedup 1.0000x reference)
import functools

import jax
import jax.numpy as jnp
from jax.experimental import pallas as pl
from jax.experimental.pallas import tpu as pltpu

_B_TOP = 8
_SCALE = 0.25

_SORT8 = [(0, 1), (2, 3), (4, 5), (6, 7),
          (0, 2), (1, 3), (4, 6), (5, 7),
          (1, 2), (5, 6), (0, 4), (3, 7),
          (1, 5), (2, 6),
          (1, 4), (3, 6),
          (2, 4), (3, 5),
          (3, 4)]

_BITONIC8 = [(0, 4), (1, 5), (2, 6), (3, 7),
             (0, 2), (1, 3), (4, 6), (5, 7),
             (0, 1), (2, 3), (4, 5), (6, 7)]


def _apply_net(v, net):
    v = list(v)
    for i, j in net:
        hi = jnp.maximum(v[i], v[j])
        lo = jnp.minimum(v[i], v[j])
        v[i], v[j] = hi, lo
    return v


def _attn_kernel(q_ref, k_ref, v_ref, vl_ref,
                 wql_ref, bql_ref, wkl_ref, bkl_ref,
                 wqh_ref, bqh_ref, wkh_ref, bkh_ref,
                 out_ref, scr0_ref, scr1_ref, klowt_ref, kt_ref, qlow_ref,
                 qh2_ref, c_ref, m_ref, mblk_ref, ssum_ref, *, nb, nib):
    g = pl.program_id(0)
    ib = pl.program_id(1)
    bi = out_ref.shape[1]
    s = k_ref.shape[1]
    p0 = g % 2
    p1 = (g + 1) % 2
    even = p0 == 0

    @pl.when(g > 0)
    def _phase1():
        coef = (jnp.exp(mblk_ref[pl.ds(p1 * nib + ib, 1), :]
                        - m_ref[pl.ds(p1, 1), :])
                / ssum_ref[pl.ds(p1, 1), :])

        def _mm(scr):
            p = scr[pl.ds(ib * bi, bi), :] * coef
            out_ref[0] = jax.lax.dot_general(
                p, v_ref[0], (((1,), (0,)), ((), ())),
                preferred_element_type=jnp.float32)

        @pl.when(even)
        def _():
            _mm(scr1_ref)

        @pl.when(jnp.logical_not(even))
        def _():
            _mm(scr0_ref)

    @pl.when(jnp.logical_and(g < nb, ib == 0))
    def _init():
        qf = q_ref[0]
        qlow_ref[...] = jax.lax.dot_general(
            qf, wql_ref[...], (((1,), (1,)), ((), ())),
            preferred_element_type=jnp.float32) + bql_ref[...]
        q_proj = jax.lax.dot_general(
            qf, wqh_ref[...], (((1,), (1,)), ((), ())),
            preferred_element_type=jnp.float32) + bqh_ref[...]
        qh2_ref[...] = jax.lax.dot_general(
            q_proj, wkh_ref[...], (((1,), (0,)), ((), ())),
            preferred_element_type=jnp.float32) * _SCALE
        c_ref[...] = jnp.sum(q_proj * bkh_ref[...], axis=1,
                             keepdims=True) * _SCALE
        klowt_ref[...] = (jax.lax.dot_general(
            wkl_ref[...], k_ref[0], (((1,), (1,)), ((), ())),
            preferred_element_type=jnp.float32) + bkl_ref[...]) * _SCALE
        kt_ref[...] = jnp.transpose(k_ref[0], (1, 0))
        m_ref[pl.ds(p0, 1), :] = jnp.full((1, s), -jnp.inf, jnp.float32)
        ssum_ref[pl.ds(p0, 1), :] = jnp.zeros((1, s), jnp.float32)

    @pl.when(g < nb)
    def _phase0():
        q_low = qlow_ref[pl.ds(ib * bi, bi), :]
        s_low = jax.lax.dot_general(
            q_low, klowt_ref[...], (((1,), (0,)), ((), ())),
            preferred_element_type=jnp.float32)
        vl = jnp.clip(vl_ref[0], 0, s - 1)
        row_ids = ib * bi + jax.lax.broadcasted_iota(jnp.int32, (bi, s), 0)
        vmask = row_ids == vl
        s_masked = jnp.where(vmask, -jnp.inf, s_low)

        nsl = s // 128
        sl = [s_masked[:, t * 128:(t + 1) * 128] for t in range(nsl)]
        a = _apply_net(sl[:8], _SORT8)
        b_ = _apply_net(sl[8:], _SORT8)
        v = [jnp.maximum(a[i], b_[7 - i]) for i in range(8)]
        v = _apply_net(v, _BITONIC8)
        for r in range(1, _B_TOP):
            m = v[0]
            for t in range(1, r):
                m = jnp.maximum(m, v[t])
            m = jnp.max(m, axis=1, keepdims=True)
            for t in range(r):
                v[t] = jnp.where(v[t] == m, -jnp.inf, v[t])
        m = v[0]
        for t in range(1, _B_TOP):
            m = jnp.maximum(m, v[t])
        t8 = jnp.max(m, axis=1, keepdims=True)
        sel = s_masked >= t8

        qh2 = qh2_ref[pl.ds(ib * bi, bi), :]
        c = c_ref[pl.ds(ib * bi, bi), :]
        s_high = jax.lax.dot_general(
            qh2, kt_ref[...], (((1,), (0,)), ((), ())),
            preferred_element_type=jnp.float32) + c

        sel2 = jnp.logical_and(sel, jnp.logical_not(vmask))
        final = jnp.where(sel2, s_high, s_masked)

        m_prev = m_ref[pl.ds(p0, 1), :]
        m_new = jnp.maximum(m_prev, jnp.max(final, axis=0, keepdims=True))
        e = jnp.exp(final - m_new)

        @pl.when(even)
        def _():
            scr0_ref[pl.ds(ib * bi, bi), :] = e

        @pl.when(jnp.logical_not(even))
        def _():
            scr1_ref[pl.ds(ib * bi, bi), :] = e

        ssum_ref[pl.ds(p0, 1), :] = (
            ssum_ref[pl.ds(p0, 1), :] * jnp.exp(m_prev - m_new)
            + jax.lax.dot_general(
                jnp.ones((1, bi), jnp.float32), e,
                (((1,), (0,)), ((), ())),
                preferred_element_type=jnp.float32))
        m_ref[pl.ds(p0, 1), :] = m_new
        mblk_ref[pl.ds(p0 * nib + ib, 1), :] = m_new


def kernel(queries, keys, values, valid_lens, Wq_low, bq_low, Wk_low, bk_low,
           Wq_high, bq_high, Wk_high, bk_high):
    b, s, hd = queries.shape
    dl = Wq_low.shape[0]
    bi = 512
    nib = s // bi
    vl3 = valid_lens.reshape(b, 1, s)
    bql = bq_low.reshape(1, dl)
    bkl = bk_low.reshape(dl, 1)
    bqh = bq_high.reshape(1, dl)
    bkh = bk_high.reshape(1, dl)

    w_spec = pl.BlockSpec((dl, hd), lambda g, ib: (0, 0))
    b_row = pl.BlockSpec((1, dl), lambda g, ib: (0, 0))

    return pl.pallas_call(
        functools.partial(_attn_kernel, nb=b, nib=nib),
        grid=(b + 1, nib),
        in_specs=[
            pl.BlockSpec((1, s, hd), lambda g, ib: (jnp.minimum(g, b - 1), 0, 0)),
            pl.BlockSpec((1, s, hd), lambda g, ib: (jnp.minimum(g, b - 1), 0, 0)),
            pl.BlockSpec((1, s, hd), lambda g, ib: (jnp.maximum(g - 1, 0),
                                                    0, 0)),
            pl.BlockSpec((1, 1, s), lambda g, ib: (jnp.minimum(g, b - 1), 0, 0)),
            w_spec, b_row, w_spec,
            pl.BlockSpec((dl, 1), lambda g, ib: (0, 0)),
            w_spec, b_row, w_spec, b_row,
        ],
        out_specs=pl.BlockSpec(
            (1, bi, hd),
            lambda g, ib: (jnp.maximum(g - 1, 0),
                           jnp.where(g == 0, 0, ib), 0)),
        out_shape=jax.ShapeDtypeStruct((b, s, hd), jnp.float32),
        scratch_shapes=[
            pltpu.VMEM((s, s), jnp.float32),
            pltpu.VMEM((s, s), jnp.float32),
            pltpu.VMEM((dl, s), jnp.float32),
            pltpu.VMEM((hd, s), jnp.float32),
            pltpu.VMEM((s, dl), jnp.float32),
            pltpu.VMEM((s, hd), jnp.float32),
            pltpu.VMEM((s, 1), jnp.float32),
            pltpu.VMEM((2, s), jnp.float32),
            pltpu.VMEM((2 * nib, s), jnp.float32),
            pltpu.VMEM((2, s), jnp.float32),
        ],
        compiler_params=pltpu.CompilerParams(
            dimension_semantics=("arbitrary", "arbitrary")),
    )(queries, keys, values, vl3, Wq_low, bql, Wk_low, bkl,
      Wq_high, bqh, Wk_high, bkh)


# speedup vs baseline: 1.0574x; 1.0574x over previous
import jax
import jax.numpy as jnp
from jax.experimental import pallas as pl
from jax.experimental.pallas import tpu as pltpu

_B_TOP = 8
_SCALE = 0.25

_SORT8 = [(0, 1), (2, 3), (4, 5), (6, 7),
          (0, 2), (1, 3), (4, 6), (5, 7),
          (1, 2), (5, 6), (0, 4), (3, 7),
          (1, 5), (2, 6),
          (1, 4), (3, 6),
          (2, 4), (3, 5),
          (3, 4)]

_BITONIC8 = [(0, 4), (1, 5), (2, 6), (3, 7),
             (0, 2), (1, 3), (4, 6), (5, 7),
             (0, 1), (2, 3), (4, 5), (6, 7)]


def _apply_net(v, net):
    v = list(v)
    for i, j in net:
        hi = jnp.maximum(v[i], v[j])
        lo = jnp.minimum(v[i], v[j])
        v[i], v[j] = hi, lo
    return v


def _attn_kernel(q_ref, k_ref, v_ref, vl_ref,
                 wql_ref, bql_ref, wkl_ref, bkl_ref,
                 wqh_ref, bqh_ref, wkh_ref, bkh_ref,
                 out_ref, scr_ref, klowt_ref, kt_ref, qlow_ref, qh2_ref,
                 c_ref, m_ref, mblk_ref, ssum_ref):
    ph = pl.program_id(1)
    ib = pl.program_id(2)
    bi = out_ref.shape[1]
    s = k_ref.shape[1]

    @pl.when(jnp.logical_and(ph == 0, ib == 0))
    def _init():
        qf = q_ref[0]
        qlow_ref[...] = jax.lax.dot_general(
            qf, wql_ref[...], (((1,), (1,)), ((), ())),
            preferred_element_type=jnp.float32) + bql_ref[...]
        q_proj = jax.lax.dot_general(
            qf, wqh_ref[...], (((1,), (1,)), ((), ())),
            preferred_element_type=jnp.float32) + bqh_ref[...]
        qh2_ref[...] = (jax.lax.dot_general(
            q_proj, wkh_ref[...], (((1,), (0,)), ((), ())),
            preferred_element_type=jnp.float32)
            * _SCALE).astype(jnp.bfloat16)
        c_ref[...] = jnp.sum(q_proj * bkh_ref[...], axis=1,
                             keepdims=True) * _SCALE
        klowt_ref[...] = (jax.lax.dot_general(
            wkl_ref[...], k_ref[0], (((1,), (1,)), ((), ())),
            preferred_element_type=jnp.float32) + bkl_ref[...]) * _SCALE
        kt_ref[...] = jnp.transpose(k_ref[0], (1, 0)).astype(jnp.bfloat16)
        m_ref[...] = jnp.full_like(m_ref[...], -jnp.inf)
        ssum_ref[...] = jnp.zeros_like(ssum_ref[...])

    @pl.when(ph == 0)
    def _phase0():
        q_low = qlow_ref[pl.ds(ib * bi, bi), :]
        s_low = jax.lax.dot_general(
            q_low, klowt_ref[...], (((1,), (0,)), ((), ())),
            preferred_element_type=jnp.float32)
        vl = jnp.clip(vl_ref[0], 0, s - 1)
        row_ids = ib * bi + jax.lax.broadcasted_iota(jnp.int32, (bi, s), 0)
        vmask = row_ids == vl
        s_masked = jnp.where(vmask, -jnp.inf, s_low)

        nsl = s // 128
        sl = [s_masked[:, t * 128:(t + 1) * 128] for t in range(nsl)]
        a = _apply_net(sl[:8], _SORT8)
        b_ = _apply_net(sl[8:], _SORT8)
        v = [jnp.maximum(a[i], b_[7 - i]) for i in range(8)]
        v = _apply_net(v, _BITONIC8)
        for r in range(1, _B_TOP):
            m = v[0]
            for t in range(1, r):
                m = jnp.maximum(m, v[t])
            m = jnp.max(m, axis=1, keepdims=True)
            for t in range(r):
                v[t] = jnp.where(v[t] == m, -jnp.inf, v[t])
        m = v[0]
        for t in range(1, _B_TOP):
            m = jnp.maximum(m, v[t])
        t8 = jnp.max(m, axis=1, keepdims=True)
        sel = s_masked >= t8

        qh2 = qh2_ref[pl.ds(ib * bi, bi), :]
        c = c_ref[pl.ds(ib * bi, bi), :]
        s_high = jax.lax.dot_general(
            qh2, kt_ref[...], (((1,), (0,)), ((), ())),
            preferred_element_type=jnp.float32) + c

        sel2 = jnp.logical_and(sel, jnp.logical_not(vmask))
        final = jnp.where(sel2, s_high, s_masked)

        m_new = jnp.maximum(m_ref[...],
                            jnp.max(final, axis=0, keepdims=True))
        e = jnp.exp(final - m_new)
        scr_ref[pl.ds(ib * bi, bi), :] = e
        ssum_ref[...] = (ssum_ref[...] * jnp.exp(m_ref[...] - m_new)
                         + jax.lax.dot_general(
                             jnp.ones((1, bi), jnp.float32), e,
                             (((1,), (0,)), ((), ())),
                             preferred_element_type=jnp.float32))
        m_ref[...] = m_new
        mblk_ref[pl.ds(ib, 1), :] = m_new

    @pl.when(ph == 1)
    def _phase1():
        coef = (jnp.exp(mblk_ref[pl.ds(ib, 1), :] - m_ref[...])
                / ssum_ref[...])
        p = scr_ref[pl.ds(ib * bi, bi), :] * coef
        out_ref[0] = jax.lax.dot_general(
            p, v_ref[0], (((1,), (0,)), ((), ())),
            preferred_element_type=jnp.float32)


def kernel(queries, keys, values, valid_lens, Wq_low, bq_low, Wk_low, bk_low,
           Wq_high, bq_high, Wk_high, bk_high):
    b, s, hd = queries.shape
    dl = Wq_low.shape[0]
    bi = 512
    nib = s // bi
    vl3 = valid_lens.reshape(b, 1, s)
    bql = bq_low.reshape(1, dl)
    bkl = bk_low.reshape(dl, 1)
    bqh = bq_high.reshape(1, dl)
    bkh = bk_high.reshape(1, dl)

    w_spec = pl.BlockSpec((dl, hd), lambda bb, ph, ib: (0, 0))

    return pl.pallas_call(
        _attn_kernel,
        grid=(b, 2, nib),
        in_specs=[
            pl.BlockSpec((1, s, hd), lambda bb, ph, ib: (bb, 0, 0)),
            pl.BlockSpec((1, s, hd), lambda bb, ph, ib: (bb, 0, 0)),
            pl.BlockSpec((1, s, hd), lambda bb, ph, ib: (bb, 0, 0)),
            pl.BlockSpec((1, 1, s), lambda bb, ph, ib: (bb, 0, 0)),
            w_spec,
            pl.BlockSpec((1, dl), lambda bb, ph, ib: (0, 0)),
            w_spec,
            pl.BlockSpec((dl, 1), lambda bb, ph, ib: (0, 0)),
            w_spec,
            pl.BlockSpec((1, dl), lambda bb, ph, ib: (0, 0)),
            w_spec,
            pl.BlockSpec((1, dl), lambda bb, ph, ib: (0, 0)),
        ],
        out_specs=pl.BlockSpec(
            (1, bi, hd),
            lambda bb, ph, ib: (bb, jnp.where(ph == 1, ib, 0), 0)),
        out_shape=jax.ShapeDtypeStruct((b, s, hd), jnp.float32),
        scratch_shapes=[
            pltpu.VMEM((s, s), jnp.float32),
            pltpu.VMEM((dl, s), jnp.float32),
            pltpu.VMEM((hd, s), jnp.bfloat16),
            pltpu.VMEM((s, dl), jnp.float32),
            pltpu.VMEM((s, hd), jnp.bfloat16),
            pltpu.VMEM((s, 1), jnp.float32),
            pltpu.VMEM((1, s), jnp.float32),
            pltpu.VMEM((nib, s), jnp.float32),
            pltpu.VMEM((1, s), jnp.float32),
        ],
        compiler_params=pltpu.CompilerParams(
            dimension_semantics=("arbitrary", "arbitrary", "arbitrary")),
    )(queries, keys, values, vl3, Wq_low, bql, Wk_low, bkl,
      Wq_high, bqh, Wk_high, bkh)


# speedup vs baseline: 1.1341x; 1.0725x over previous
import jax
import jax.numpy as jnp
from jax.experimental import pallas as pl
from jax.experimental.pallas import tpu as pltpu

_B_TOP = 8
_SCALE = 0.25

_SORT8 = [(0, 1), (2, 3), (4, 5), (6, 7),
          (0, 2), (1, 3), (4, 6), (5, 7),
          (1, 2), (5, 6), (0, 4), (3, 7),
          (1, 5), (2, 6),
          (1, 4), (3, 6),
          (2, 4), (3, 5),
          (3, 4)]

_BITONIC8 = [(0, 4), (1, 5), (2, 6), (3, 7),
             (0, 2), (1, 3), (4, 6), (5, 7),
             (0, 1), (2, 3), (4, 5), (6, 7)]


def _apply_net(v, net):
    v = list(v)
    for i, j in net:
        hi = jnp.maximum(v[i], v[j])
        lo = jnp.minimum(v[i], v[j])
        v[i], v[j] = hi, lo
    return v


def _attn_kernel(q_ref, k_ref, v_ref, vl_ref,
                 wql_ref, bql_ref, wkl_ref, bkl_ref,
                 wqh_ref, bqh_ref, wkh_ref, bkh_ref,
                 out_ref, scr_ref, klowt_ref, kt_ref, qlow_ref, qh2_ref,
                 c_ref, m_ref, mblk_ref, ssum_ref):
    ph = pl.program_id(1)
    ib = pl.program_id(2)
    bi = out_ref.shape[1]
    s = k_ref.shape[1]

    @pl.when(jnp.logical_and(ph == 0, ib == 0))
    def _init():
        qf = q_ref[0]
        qlow_ref[...] = jax.lax.dot_general(
            qf, wql_ref[...], (((1,), (1,)), ((), ())),
            preferred_element_type=jnp.float32) + bql_ref[...]
        q_proj = jax.lax.dot_general(
            qf, wqh_ref[...], (((1,), (1,)), ((), ())),
            preferred_element_type=jnp.float32) + bqh_ref[...]
        qh2_ref[...] = (jax.lax.dot_general(
            q_proj, wkh_ref[...], (((1,), (0,)), ((), ())),
            preferred_element_type=jnp.float32)
            * _SCALE).astype(jnp.bfloat16)
        c_ref[...] = jnp.sum(q_proj * bkh_ref[...], axis=1,
                             keepdims=True) * _SCALE
        klowt_ref[...] = (jax.lax.dot_general(
            wkl_ref[...], k_ref[0], (((1,), (1,)), ((), ())),
            preferred_element_type=jnp.float32) + bkl_ref[...]) * _SCALE
        kt_ref[...] = jnp.transpose(k_ref[0], (1, 0)).astype(jnp.bfloat16)
        m_ref[...] = jnp.full_like(m_ref[...], -jnp.inf)
        ssum_ref[...] = jnp.zeros_like(ssum_ref[...])

    @pl.when(ph == 0)
    def _phase0():
        q_low = qlow_ref[pl.ds(ib * bi, bi), :]
        s_low = jax.lax.dot_general(
            q_low, klowt_ref[...], (((1,), (0,)), ((), ())),
            preferred_element_type=jnp.float32)
        vl = jnp.clip(vl_ref[0], 0, s - 1)
        row_ids = ib * bi + jax.lax.broadcasted_iota(jnp.int32, (bi, s), 0)
        vmask = row_ids == vl
        s_masked = jnp.where(vmask, -jnp.inf, s_low)

        nsl = s // 128
        sl = [s_masked[:, t * 128:(t + 1) * 128] for t in range(nsl)]
        a = _apply_net(sl[:8], _SORT8)
        b_ = _apply_net(sl[8:], _SORT8)
        v = [jnp.maximum(a[i], b_[7 - i]) for i in range(8)]
        v = _apply_net(v, _BITONIC8)
        for r in range(1, _B_TOP):
            m = v[0]
            for t in range(1, r):
                m = jnp.maximum(m, v[t])
            m = jnp.max(m, axis=1, keepdims=True)
            for t in range(r):
                v[t] = jnp.where(v[t] == m, -jnp.inf, v[t])
        m = v[0]
        for t in range(1, _B_TOP):
            m = jnp.maximum(m, v[t])
        t8 = jnp.max(m, axis=1, keepdims=True)
        sel = s_masked >= t8

        qh2 = qh2_ref[pl.ds(ib * bi, bi), :]
        c = c_ref[pl.ds(ib * bi, bi), :]
        s_high = jax.lax.dot_general(
            qh2, kt_ref[...], (((1,), (0,)), ((), ())),
            preferred_element_type=jnp.float32) + c

        sel2 = jnp.logical_and(sel, jnp.logical_not(vmask))
        final = jnp.where(sel2, s_high, s_masked)

        m_new = jnp.maximum(m_ref[...],
                            jnp.max(final, axis=0, keepdims=True))
        e = jnp.exp(final - m_new)
        scr_ref[pl.ds(ib * bi, bi), :] = e
        ssum_ref[...] = (ssum_ref[...] * jnp.exp(m_ref[...] - m_new)
                         + jax.lax.dot_general(
                             jnp.ones((1, bi), jnp.float32), e,
                             (((1,), (0,)), ((), ())),
                             preferred_element_type=jnp.float32))
        m_ref[...] = m_new
        mblk_ref[pl.ds(ib, 1), :] = m_new

    @pl.when(ph == 1)
    def _phase1():
        coef = (jnp.exp(mblk_ref[pl.ds(ib, 1), :] - m_ref[...])
                / ssum_ref[...])
        p = scr_ref[pl.ds(ib * bi, bi), :] * coef
        out_ref[0] = jax.lax.dot_general(
            p, v_ref[0], (((1,), (0,)), ((), ())),
            preferred_element_type=jnp.float32)


def kernel(queries, keys, values, valid_lens, Wq_low, bq_low, Wk_low, bk_low,
           Wq_high, bq_high, Wk_high, bk_high):
    b, s, hd = queries.shape
    dl = Wq_low.shape[0]
    bi = 1024
    nib = s // bi
    vl3 = valid_lens.reshape(b, 1, s)
    bql = bq_low.reshape(1, dl)
    bkl = bk_low.reshape(dl, 1)
    bqh = bq_high.reshape(1, dl)
    bkh = bk_high.reshape(1, dl)

    w_spec = pl.BlockSpec((dl, hd), lambda bb, ph, ib: (0, 0))

    return pl.pallas_call(
        _attn_kernel,
        grid=(b, 2, nib),
        in_specs=[
            pl.BlockSpec((1, s, hd), lambda bb, ph, ib: (bb, 0, 0)),
            pl.BlockSpec((1, s, hd), lambda bb, ph, ib: (bb, 0, 0)),
            pl.BlockSpec((1, s, hd), lambda bb, ph, ib: (bb, 0, 0)),
            pl.BlockSpec((1, 1, s), lambda bb, ph, ib: (bb, 0, 0)),
            w_spec,
            pl.BlockSpec((1, dl), lambda bb, ph, ib: (0, 0)),
            w_spec,
            pl.BlockSpec((dl, 1), lambda bb, ph, ib: (0, 0)),
            w_spec,
            pl.BlockSpec((1, dl), lambda bb, ph, ib: (0, 0)),
            w_spec,
            pl.BlockSpec((1, dl), lambda bb, ph, ib: (0, 0)),
        ],
        out_specs=pl.BlockSpec(
            (1, bi, hd),
            lambda bb, ph, ib: (bb, jnp.where(ph == 1, ib, 0), 0)),
        out_shape=jax.ShapeDtypeStruct((b, s, hd), jnp.float32),
        scratch_shapes=[
            pltpu.VMEM((s, s), jnp.float32),
            pltpu.VMEM((dl, s), jnp.float32),
            pltpu.VMEM((hd, s), jnp.bfloat16),
            pltpu.VMEM((s, dl), jnp.float32),
            pltpu.VMEM((s, hd), jnp.bfloat16),
            pltpu.VMEM((s, 1), jnp.float32),
            pltpu.VMEM((1, s), jnp.float32),
            pltpu.VMEM((nib, s), jnp.float32),
            pltpu.VMEM((1, s), jnp.float32),
        ],
        compiler_params=pltpu.CompilerParams(
            dimension_semantics=("arbitrary", "arbitrary", "arbitrary")),
    )(queries, keys, values, vl3, Wq_low, bql, Wk_low, bkl,
      Wq_high, bqh, Wk_high, bkh)
